# baseline (speedup 1.0000x reference)
import functools

import jax
import jax.numpy as jnp
from jax import lax
from jax.experimental import pallas as pl
from jax.experimental.pallas import tpu as pltpu
from jax.experimental.pallas import tpu_sc as plsc

N = 10000
E = 160000
D = 300
H = 600
L = 5
G = 64
OUT = 2048

PD = 128
NPART = 3
NP_ = 10240
NC = 2
NS = 16
CH = 64
EPT = 10240
NCHUNK = EPT // CH
SEG = 20
ZROWS = NP_ // NS
B_TC = 512
C_HD = 1280


def _gather_scatter_segment(table, idx, acc, bufs, gsems, ssems):
    n = SEG
    pltpu.async_copy(table.at[idx.at[0, 0]], bufs[0], gsems[0])
    pltpu.async_copy(table.at[idx.at[1, 0]], bufs[1], gsems[1])

    def quad(i, _):
        for u in range(4):
            k = 4 * i + u
            v = (u + 2) % 4
            pltpu.make_async_copy(table.at[idx.at[k, 0]], bufs[u],
                                  gsems[u]).wait()
            pltpu.async_copy(bufs[u], acc.at[idx.at[k, 1]], ssems[u], add=True)

            @pl.when(k >= 2)
            def _():
                pltpu.make_async_copy(bufs[v], acc.at[idx.at[k - 2, 1]],
                                      ssems[v]).wait()

            @pl.when(k + 2 < n)
            def _():
                pltpu.async_copy(table.at[idx.at[k + 2, 0]], bufs[v], gsems[v])
        return 0

    lax.fori_loop(0, n // 4, quad, 0)
    pltpu.make_async_copy(bufs[2], acc.at[idx.at[n - 2, 1]], ssems[2]).wait()
    pltpu.make_async_copy(bufs[3], acc.at[idx.at[n - 1, 1]], ssems[3]).wait()


def _sc_body(h_hbm, idx_hbm, zero_hbm, agg_hbm,
             i0, i1, b0, b1, b2, b3, acc,
             u0, u1, g0, g1, g2, g3, s0, s1, s2, s3):
    c = lax.axis_index("c")
    s = lax.axis_index("s")
    rows = pl.ds(s * ZROWS, ZROWS)
    ibufs = (i0, i1)
    isems = (u0, u1)
    bufs = (b0, b1, b2, b3)
    gsems = (g0, g1, g2, g3)
    ssems = (s0, s1, s2, s3)

    table = h_hbm.at[c]
    pltpu.sync_copy(zero_hbm, acc.at[rows])
    plsc.subcore_barrier()
    n = NCHUNK // SEG

    def slab(j):
        return idx_hbm.at[s, pl.ds(j * SEG, SEG)]

    pltpu.async_copy(slab(0), ibufs[0], isems[0])
    for j in range(n):
        pltpu.make_async_copy(slab(j), ibufs[j % 2], isems[j % 2]).wait()
        if j + 1 < n:
            pltpu.async_copy(slab(j + 1), ibufs[(j + 1) % 2],
                             isems[(j + 1) % 2])
        _gather_scatter_segment(table, ibufs[j % 2], acc, bufs, gsems, ssems)
    plsc.subcore_barrier()
    pltpu.sync_copy(acc.at[rows], agg_hbm.at[c].at[rows])


@functools.cache
def _sc_msg_kernel():
    return pl.kernel(
        _sc_body,
        out_type=(
            jax.ShapeDtypeStruct((2, NP_, PD), jnp.float32),
        ),
        mesh=plsc.VectorSubcoreMesh(
            core_axis_name="c", subcore_axis_name="s",
            num_cores=NC, num_subcores=NS),
        scratch_types=(
            [pltpu.VMEM((SEG, 2, CH), jnp.int32)] * 2
            + [pltpu.VMEM((CH, PD), jnp.float32)] * 4
            + [pltpu.VMEM_SHARED((NP_, PD), jnp.float32)]
            + [pltpu.SemaphoreType.DMA] * 10
        ),
    )


def _sc_msg(h, idx_t, zrows):
    return _sc_msg_kernel()(h, idx_t, zrows)[0]


B_ADJ = 512
KC = 2048


def _adj_body(a_ref, h_ref, out_ref, acc):
    k = pl.program_id(1)

    @pl.when(k == 0)
    def _():
        acc[...] = jnp.zeros_like(acc)

    acc[...] += jnp.dot(a_ref[...], h_ref[...],
                        preferred_element_type=jnp.float32)

    @pl.when(k == NP_ // KC - 1)
    def _():
        out_ref[...] = acc[...]


def _adj_call(a, h2):
    return pl.pallas_call(
        _adj_body,
        grid=(NP_ // B_ADJ, NP_ // KC),
        in_specs=[
            pl.BlockSpec((B_ADJ, KC), lambda i, k: (i, k)),
            pl.BlockSpec((KC, PD), lambda i, k: (k, 0)),
        ],
        out_specs=pl.BlockSpec((B_ADJ, PD), lambda i, k: (i, 0)),
        out_shape=jax.ShapeDtypeStruct((NP_, PD), jnp.float32),
        scratch_shapes=[pltpu.VMEM((B_ADJ, PD), jnp.float32)],
    )(a, h2)


def _mlp_body(scale_ref, h_ref, agg_ref, agg2_ref, w1_ref, b1_ref, w2_ref,
              b2_ref, out_ref):
    scale = scale_ref[0, 0]
    hin = [scale * h_ref[0] + agg_ref[0],
           scale * h_ref[1] + agg_ref[1],
           scale * h_ref[2] + agg2_ref[...]]
    t = b1_ref[...]
    for p in range(NPART):
        t = t + jnp.dot(hin[p], w1_ref[p], preferred_element_type=jnp.float32)
    t = jnp.maximum(t, 0.0)
    for p in range(NPART):
        out_ref[p] = jnp.maximum(
            jnp.dot(t, w2_ref[p], preferred_element_type=jnp.float32)
            + b2_ref[p][None, :], 0.0)


def _mlp_call(scale, h, agg, agg2, w1, b1, w2, b2):
    return pl.pallas_call(
        _mlp_body,
        grid=(NP_ // B_TC,),
        in_specs=[
            pl.BlockSpec(memory_space=pltpu.SMEM),
            pl.BlockSpec((NPART, B_TC, PD), lambda i: (0, i, 0)),
            pl.BlockSpec((2, B_TC, PD), lambda i: (0, i, 0)),
            pl.BlockSpec((B_TC, PD), lambda i: (i, 0)),
            pl.BlockSpec((NPART, PD, H), lambda i: (0, 0, 0)),
            pl.BlockSpec((1, H), lambda i: (0, 0)),
            pl.BlockSpec((NPART, H, PD), lambda i: (0, 0, 0)),
            pl.BlockSpec((NPART, PD), lambda i: (0, 0)),
        ],
        out_specs=pl.BlockSpec((NPART, B_TC, PD), lambda i: (0, i, 0)),
        out_shape=jax.ShapeDtypeStruct((NPART, NP_, PD), jnp.float32),
    )(scale, h, agg, agg2, w1, b1, w2, b2)


def _head_body(h_ref, p_ref, wh_ref, bh_ref, out_ref, accs):
    i = pl.program_id(0)

    @pl.when(i == 0)
    def _():
        accs[...] = jnp.zeros_like(accs)

    for p in range(NPART):
        accs[p] += jnp.dot(p_ref[...], h_ref[p],
                           preferred_element_type=jnp.float32)

    @pl.when(i == NP_ // C_HD - 1)
    def _():
        cnt = accs[NPART - 1][:, PD - 1:PD]
        inv = 1.0 / jnp.maximum(cnt, 1.0)
        out = bh_ref[...]
        for p in range(NPART):
            out = out + jnp.dot(accs[p] * inv, wh_ref[p],
                                preferred_element_type=jnp.float32)
        out_ref[...] = out


def _head_call(h, p, wh, bh):
    return pl.pallas_call(
        _head_body,
        grid=(NP_ // C_HD,),
        in_specs=[
            pl.BlockSpec((NPART, C_HD, PD), lambda i: (0, i, 0)),
            pl.BlockSpec((G, C_HD), lambda i: (0, i)),
            pl.BlockSpec((NPART, PD, OUT), lambda i: (0, 0, 0)),
            pl.BlockSpec((1, OUT), lambda i: (0, 0)),
        ],
        out_specs=pl.BlockSpec((G, OUT), lambda i: (0, 0)),
        out_shape=jax.ShapeDtypeStruct((G, OUT), jnp.float32),
        scratch_shapes=[
            pltpu.VMEM((NPART, G, PD), jnp.float32),
        ],
    )(h, p, wh, bh)


def _part_pad(a, ncols_axis=-1):
    pads = [(0, 0)] * a.ndim
    pads[ncols_axis] = (0, NPART * PD - D)
    ap = jnp.pad(a, pads)
    return ap


def kernel(x, edge_index, graph_ids, W1, b1, W2, b2, eps, W_head, b_head):
    f32 = jnp.float32
    src = edge_index[0].astype(jnp.int32)
    dst = edge_index[1].astype(jnp.int32)

    src_t = jnp.full((NS, EPT), NP_ - 1, jnp.int32).at[:, :E // NS].set(
        src.reshape(NS, E // NS)).reshape(NS, NCHUNK, CH)
    dst_t = jnp.full((NS, EPT), N, jnp.int32).at[:, :E // NS].set(
        dst.reshape(NS, E // NS)).reshape(NS, NCHUNK, CH)
    idx_t = jnp.stack([src_t, dst_t], axis=2)

    xp = _part_pad(x)
    h = jnp.zeros((NPART, NP_, PD), f32)
    for p in range(NPART):
        h = h.at[p, :N].set(xp[:, p * PD:(p + 1) * PD])

    W1p = _part_pad(W1, 1).reshape(L, NPART, PD, H)
    W2p = _part_pad(W2).reshape(L, H, NPART, PD).transpose(0, 2, 1, 3)
    b2p = _part_pad(b2).reshape(L, NPART, PD)
    b1r = b1.reshape(L, 1, H)
    scales = (1.0 + eps).astype(f32).reshape(L, 1, 1)
    zrows = jnp.zeros((ZROWS, PD), f32)

    amat = jnp.zeros((NP_, NP_), jnp.bfloat16).at[dst, src].add(
        jnp.bfloat16(1.0))

    for l in range(L):
        agg = _sc_msg(h, idx_t, zrows)
        agg2 = _adj_call(amat, h[2].astype(jnp.bfloat16))
        h = _mlp_call(scales[l], h, agg, agg2,
                      W1p[l], b1r[l], W2p[l], b2p[l])

    onehot = (graph_ids[None, :] == jnp.arange(G, dtype=graph_ids.dtype)[:, None])
    pmat = jnp.zeros((G, NP_), f32).at[:, :N].set(onehot.astype(f32))
    hh = h.at[NPART - 1, :, PD - 1].set(1.0)
    whp = _part_pad(W_head, 0).reshape(NPART, PD, OUT)
    return _head_call(hh, pmat, whp, b_head.reshape(1, OUT))


# speedup vs baseline: 1.3199x; 1.3199x over previous
import functools

import jax
import jax.numpy as jnp
from jax import lax
from jax.experimental import pallas as pl
from jax.experimental.pallas import tpu as pltpu
from jax.experimental.pallas import tpu_sc as plsc

N = 10000
E = 160000
D = 300
H = 600
L = 5
G = 64
OUT = 2048

PD = 128
NPART = 3
NP_ = 10240
NC = 2
NS = 16
CH = 64
EPT = 10240
NCHUNK = EPT // CH
SEG = 20
ZROWS = NP_ // NS
B_TC = 512
C_HD = 1280


def _gather_scatter_segment(table, idx, acc, bufs, gsems, ssems):
    n = SEG
    pltpu.async_copy(table.at[idx.at[0, 0]], bufs[0], gsems[0])
    pltpu.async_copy(table.at[idx.at[1, 0]], bufs[1], gsems[1])

    def quad(i, _):
        for u in range(4):
            k = 4 * i + u
            v = (u + 2) % 4
            pltpu.make_async_copy(table.at[idx.at[k, 0]], bufs[u],
                                  gsems[u]).wait()
            pltpu.async_copy(bufs[u], acc.at[idx.at[k, 1]], ssems[u], add=True)

            @pl.when(k >= 2)
            def _():
                pltpu.make_async_copy(bufs[v], acc.at[idx.at[k - 2, 1]],
                                      ssems[v]).wait()

            @pl.when(k + 2 < n)
            def _():
                pltpu.async_copy(table.at[idx.at[k + 2, 0]], bufs[v], gsems[v])
        return 0

    lax.fori_loop(0, n // 4, quad, 0)
    pltpu.make_async_copy(bufs[2], acc.at[idx.at[n - 2, 1]], ssems[2]).wait()
    pltpu.make_async_copy(bufs[3], acc.at[idx.at[n - 1, 1]], ssems[3]).wait()


def _sc_body(h_hbm, idx_hbm, zero_hbm, agg_hbm, agg2_hbm,
             i0, i1, b0, b1, b2, b3, acc,
             u0, u1, g0, g1, g2, g3, s0, s1, s2, s3):
    c = lax.axis_index("c")
    s = lax.axis_index("s")
    rows = pl.ds(s * ZROWS, ZROWS)
    ibufs = (i0, i1)
    isems = (u0, u1)
    bufs = (b0, b1, b2, b3)
    gsems = (g0, g1, g2, g3)
    ssems = (s0, s1, s2, s3)

    def run_pass(table, segs, preds, out_view):
        pltpu.sync_copy(zero_hbm, acc.at[rows])
        plsc.subcore_barrier()
        n = len(segs)

        def slab(j):
            return idx_hbm.at[s, pl.ds(segs[j] * SEG, SEG)]

        def guarded(j, fn):
            def _body():
                fn()
            if preds is None:
                _body()
            else:
                pl.when(preds[j])(_body)

        guarded(0, lambda: pltpu.async_copy(slab(0), ibufs[0], isems[0]))
        for j in range(n):
            guarded(j, lambda j=j: pltpu.make_async_copy(
                slab(j), ibufs[j % 2], isems[j % 2]).wait())
            if j + 1 < n:
                guarded(j + 1, lambda j=j: pltpu.async_copy(
                    slab(j + 1), ibufs[(j + 1) % 2], isems[(j + 1) % 2]))
            guarded(j, lambda j=j: _gather_scatter_segment(
                table, ibufs[j % 2], acc, bufs, gsems, ssems))
        plsc.subcore_barrier()
        pltpu.sync_copy(acc.at[rows], out_view.at[rows])

    run_pass(h_hbm.at[c], list(range(8)), None, agg_hbm.at[c])

    run_pass(h_hbm.at[2], [6 * c + j for j in range(6)],
             [(c == 0) | (j < 2) for j in range(6)], agg2_hbm.at[c])


@functools.cache
def _sc_msg_kernel():
    return pl.kernel(
        _sc_body,
        out_type=(
            jax.ShapeDtypeStruct((2, NP_, PD), jnp.float32),
            jax.ShapeDtypeStruct((2, NP_, PD), jnp.float32),
        ),
        mesh=plsc.VectorSubcoreMesh(
            core_axis_name="c", subcore_axis_name="s",
            num_cores=NC, num_subcores=NS),
        scratch_types=(
            [pltpu.VMEM((SEG, 2, CH), jnp.int32)] * 2
            + [pltpu.VMEM((CH, PD), jnp.float32)] * 4
            + [pltpu.VMEM_SHARED((NP_, PD), jnp.float32)]
            + [pltpu.SemaphoreType.DMA] * 10
        ),
    )


def _sc_msg(h, idx_t, zrows):
    return _sc_msg_kernel()(h, idx_t, zrows)


def _mlp_body(scale_ref, h_ref, agg_ref, agg2_ref, w1_ref, b1_ref, w2_ref,
              b2_ref, out_ref):
    scale = scale_ref[0, 0]
    hin = [scale * h_ref[0] + agg_ref[0],
           scale * h_ref[1] + agg_ref[1],
           scale * h_ref[2] + agg2_ref[0] + agg2_ref[1]]
    t = b1_ref[...]
    for p in range(NPART):
        t = t + jnp.dot(hin[p], w1_ref[p], preferred_element_type=jnp.float32)
    t = jnp.maximum(t, 0.0)
    for p in range(NPART):
        out_ref[p] = jnp.maximum(
            jnp.dot(t, w2_ref[p], preferred_element_type=jnp.float32)
            + b2_ref[p][None, :], 0.0)


def _mlp_call(scale, h, agg, agg2, w1, b1, w2, b2):
    return pl.pallas_call(
        _mlp_body,
        grid=(NP_ // B_TC,),
        in_specs=[
            pl.BlockSpec(memory_space=pltpu.SMEM),
            pl.BlockSpec((NPART, B_TC, PD), lambda i: (0, i, 0)),
            pl.BlockSpec((2, B_TC, PD), lambda i: (0, i, 0)),
            pl.BlockSpec((2, B_TC, PD), lambda i: (0, i, 0)),
            pl.BlockSpec((NPART, PD, H), lambda i: (0, 0, 0)),
            pl.BlockSpec((1, H), lambda i: (0, 0)),
            pl.BlockSpec((NPART, H, PD), lambda i: (0, 0, 0)),
            pl.BlockSpec((NPART, PD), lambda i: (0, 0)),
        ],
        out_specs=pl.BlockSpec((NPART, B_TC, PD), lambda i: (0, i, 0)),
        out_shape=jax.ShapeDtypeStruct((NPART, NP_, PD), jnp.float32),
    )(scale, h, agg, agg2, w1, b1, w2, b2)


def _head_body(h_ref, p_ref, wh_ref, bh_ref, out_ref, accs):
    i = pl.program_id(0)

    @pl.when(i == 0)
    def _():
        accs[...] = jnp.zeros_like(accs)

    for p in range(NPART):
        accs[p] += jnp.dot(p_ref[...], h_ref[p],
                           preferred_element_type=jnp.float32)

    @pl.when(i == NP_ // C_HD - 1)
    def _():
        cnt = accs[NPART - 1][:, PD - 1:PD]
        inv = 1.0 / jnp.maximum(cnt, 1.0)
        out = bh_ref[...]
        for p in range(NPART):
            out = out + jnp.dot(accs[p] * inv, wh_ref[p],
                                preferred_element_type=jnp.float32)
        out_ref[...] = out


def _head_call(h, p, wh, bh):
    return pl.pallas_call(
        _head_body,
        grid=(NP_ // C_HD,),
        in_specs=[
            pl.BlockSpec((NPART, C_HD, PD), lambda i: (0, i, 0)),
            pl.BlockSpec((G, C_HD), lambda i: (0, i)),
            pl.BlockSpec((NPART, PD, OUT), lambda i: (0, 0, 0)),
            pl.BlockSpec((1, OUT), lambda i: (0, 0)),
        ],
        out_specs=pl.BlockSpec((G, OUT), lambda i: (0, 0)),
        out_shape=jax.ShapeDtypeStruct((G, OUT), jnp.float32),
        scratch_shapes=[
            pltpu.VMEM((NPART, G, PD), jnp.float32),
        ],
    )(h, p, wh, bh)


def _part_pad(a, ncols_axis=-1):
    pads = [(0, 0)] * a.ndim
    pads[ncols_axis] = (0, NPART * PD - D)
    ap = jnp.pad(a, pads)
    return ap


def kernel(x, edge_index, graph_ids, W1, b1, W2, b2, eps, W_head, b_head):
    f32 = jnp.float32
    src = edge_index[0].astype(jnp.int32)
    dst = edge_index[1].astype(jnp.int32)

    src_t = jnp.full((NS, EPT), NP_ - 1, jnp.int32).at[:, :E // NS].set(
        src.reshape(NS, E // NS)).reshape(NS, NCHUNK, CH)
    dst_t = jnp.full((NS, EPT), N, jnp.int32).at[:, :E // NS].set(
        dst.reshape(NS, E // NS)).reshape(NS, NCHUNK, CH)
    idx_t = jnp.stack([src_t, dst_t], axis=2)

    xp = _part_pad(x)
    h = jnp.zeros((NPART, NP_, PD), f32)
    for p in range(NPART):
        h = h.at[p, :N].set(xp[:, p * PD:(p + 1) * PD])

    W1p = _part_pad(W1, 1).reshape(L, NPART, PD, H)
    W2p = _part_pad(W2).reshape(L, H, NPART, PD).transpose(0, 2, 1, 3)
    b2p = _part_pad(b2).reshape(L, NPART, PD)
    b1r = b1.reshape(L, 1, H)
    scales = (1.0 + eps).astype(f32).reshape(L, 1, 1)
    zrows = jnp.zeros((ZROWS, PD), f32)

    for l in range(L):
        agg, agg2 = _sc_msg(h, idx_t, zrows)
        h = _mlp_call(scales[l], h, agg, agg2,
                      W1p[l], b1r[l], W2p[l], b2p[l])

    onehot = (graph_ids[None, :] == jnp.arange(G, dtype=graph_ids.dtype)[:, None])
    pmat = jnp.zeros((G, NP_), f32).at[:, :N].set(onehot.astype(f32))
    hh = h.at[NPART - 1, :, PD - 1].set(1.0)
    whp = _part_pad(W_head, 0).reshape(NPART, PD, OUT)
    return _head_call(hh, pmat, whp, b_head.reshape(1, OUT))
